# baseline (speedup 1.0000x reference)
import jax
import jax.numpy as jnp
from jax.experimental import pallas as pl
from jax.experimental.pallas import tpu as pltpu


def _copy_body(x_ref, o_ref, sem):
    copy = pltpu.make_async_copy(x_ref, o_ref, sem)
    copy.start()
    copy.wait()


def kernel(stack):
    return pl.pallas_call(
        _copy_body,
        in_specs=[pl.BlockSpec(memory_space=pl.ANY)],
        out_specs=pl.BlockSpec(memory_space=pl.ANY),
        out_shape=jax.ShapeDtypeStruct(stack.shape, stack.dtype),
        scratch_shapes=[pltpu.SemaphoreType.DMA],
    )(stack)


# speedup vs baseline: 16.2242x; 16.2242x over previous
import jax
import jax.numpy as jnp
from jax.experimental import pallas as pl
from jax.experimental.pallas import tpu as pltpu

_B = 16


def _copy_body(x_ref, o_ref):
    o_ref[...] = x_ref[...]


def kernel(stack):
    n = stack.shape[0] // _B
    return pl.pallas_call(
        _copy_body,
        grid=(n,),
        in_specs=[pl.BlockSpec((_B, 1024, 64), lambda i: (i, 0, 0))],
        out_specs=pl.BlockSpec((_B, 1024, 64), lambda i: (i, 0, 0)),
        out_shape=jax.ShapeDtypeStruct(stack.shape, stack.dtype),
    )(stack)


# speedup vs baseline: 102.3705x; 6.3098x over previous
import jax
import jax.numpy as jnp
from jax.experimental import pallas as pl

_R = 2048


def _copy_body(x_ref, o_ref):
    o_ref[...] = x_ref[...]


def kernel(stack):
    n, s, d = stack.shape
    x = stack.transpose(0, 2, 1).reshape(n * d, s)
    rows = n * d
    y = pl.pallas_call(
        _copy_body,
        grid=(rows // _R,),
        in_specs=[pl.BlockSpec((_R, s), lambda i: (i, 0))],
        out_specs=pl.BlockSpec((_R, s), lambda i: (i, 0)),
        out_shape=jax.ShapeDtypeStruct((rows, s), stack.dtype),
    )(x)
    return y.reshape(n, d, s).transpose(0, 2, 1)
